# initial kernel scaffold (unmeasured)
import jax
import jax.numpy as jnp
from jax import lax
from jax.experimental import pallas as pl
from jax.experimental.pallas import tpu as pltpu

B, H, D, BS = 32, 16, 128, 32
PAGES_LOCAL = 256
SCALE = D ** -0.5
NEG = -1e30


def _partials_body(q_ref, bt_ref, lens_ref, k_hbm, v_hbm,
                   acc_ref, m_ref, l_ref,
                   k_buf, v_buf, sems):
    i = pl.program_id(0)
    my_z = lax.axis_index("z")
    base = my_z * PAGES_LOCAL
    n = lens_ref[i]
    q = q_ref[0, 0]

    def page_step(j, carry):
        m, l, acc = carry
        p_loc = bt_ref[i, j] - base
        owned = jnp.logical_and(p_loc >= 0, p_loc < PAGES_LOCAL)
        pc = jnp.clip(p_loc, 0, PAGES_LOCAL - 1)
        ck = pltpu.make_async_copy(k_hbm.at[pc], k_buf, sems.at[0])
        cv = pltpu.make_async_copy(v_hbm.at[pc], v_buf, sems.at[1])
        ck.start()
        cv.start()
        ck.wait()
        cv.wait()
        k = k_buf[...]
        s = jnp.sum(q[None, :, :] * k, axis=-1) * SCALE
        page_max = jnp.where(owned, jnp.max(s, axis=0, keepdims=True), NEG)
        m_new = jnp.maximum(m, page_max)
        alpha = jnp.exp(m - m_new)
        pexp = jnp.where(owned, jnp.exp(s - m_new), 0.0)
        l_new = alpha * l + jnp.sum(pexp, axis=0, keepdims=True)
        v = v_buf[...]
        pv = jnp.sum(pexp[:, :, None] * v, axis=0)
        acc_new = acc * jnp.reshape(alpha, (H, 1)) + pv
        return m_new, l_new, acc_new

    m0 = jnp.full((1, H), NEG, jnp.float32)
    l0 = jnp.zeros((1, H), jnp.float32)
    a0 = jnp.zeros((H, D), jnp.float32)
    m, l, acc = lax.fori_loop(0, n, page_step, (m0, l0, a0))
    m_ref[...] = m
    l_ref[...] = l
    acc_ref[0] = acc


def _combine_body(acc_ref, m_ref, l_ref, out_ref,
                  racc_ref, rm_ref, rl_ref, send_sems, recv_sems):
    x = lax.axis_index("x")
    y = lax.axis_index("y")
    z = lax.axis_index("z")
    partner = (x, y, 1 - z)

    bsem = pltpu.get_barrier_semaphore()
    pl.semaphore_signal(bsem, inc=1, device_id=partner,
                        device_id_type=pl.DeviceIdType.MESH)
    pl.semaphore_wait(bsem, 1)

    copies = []
    for idx, (src, dst) in enumerate(
        [(acc_ref, racc_ref), (m_ref, rm_ref), (l_ref, rl_ref)]
    ):
        c = pltpu.make_async_remote_copy(
            src_ref=src, dst_ref=dst,
            send_sem=send_sems.at[idx], recv_sem=recv_sems.at[idx],
            device_id=partner, device_id_type=pl.DeviceIdType.MESH,
        )
        c.start()
        copies.append(c)
    for c in copies:
        c.wait()

    m_a = m_ref[...]
    m_b = rm_ref[...]
    mx = jnp.maximum(m_a, m_b)
    wa = jnp.exp(m_a - mx)
    wb = jnp.exp(m_b - mx)
    denom = l_ref[...] * wa + rl_ref[...] * wb
    num = acc_ref[...] * wa[:, :, None] + racc_ref[...] * wb[:, :, None]
    out_ref[:, 0, :, :] = num / denom[:, :, None]


def kernel(Q, K, V, bt, lens):
    acc, m, l = pl.pallas_call(
        _partials_body,
        grid=(B,),
        in_specs=[
            pl.BlockSpec((1, 1, H, D), lambda i: (i, 0, 0, 0)),
            pl.BlockSpec(memory_space=pltpu.SMEM),
            pl.BlockSpec(memory_space=pltpu.SMEM),
            pl.BlockSpec(memory_space=pltpu.ANY),
            pl.BlockSpec(memory_space=pltpu.ANY),
        ],
        out_specs=[
            pl.BlockSpec((1, H, D), lambda i: (i, 0, 0)),
            pl.BlockSpec((1, H), lambda i: (i, 0)),
            pl.BlockSpec((1, H), lambda i: (i, 0)),
        ],
        out_shape=[
            jax.ShapeDtypeStruct((B, H, D), jnp.float32),
            jax.ShapeDtypeStruct((B, H), jnp.float32),
            jax.ShapeDtypeStruct((B, H), jnp.float32),
        ],
        scratch_shapes=[
            pltpu.VMEM((BS, H, D), jnp.float32),
            pltpu.VMEM((BS, H, D), jnp.float32),
            pltpu.SemaphoreType.DMA((2,)),
        ],
    )(Q, bt, lens, K, V)

    return pl.pallas_call(
        _combine_body,
        in_specs=[pl.BlockSpec(memory_space=pltpu.VMEM)] * 3,
        out_specs=pl.BlockSpec(memory_space=pltpu.VMEM),
        out_shape=jax.ShapeDtypeStruct((B, 1, H, D), jnp.float32),
        scratch_shapes=[
            pltpu.VMEM((B, H, D), jnp.float32),
            pltpu.VMEM((B, H), jnp.float32),
            pltpu.VMEM((B, H), jnp.float32),
            pltpu.SemaphoreType.DMA((3,)),
            pltpu.SemaphoreType.DMA((3,)),
        ],
        compiler_params=pltpu.CompilerParams(collective_id=0),
    )(acc, m, l)


# baseline (device time: 5862471 ns/iter reference)
import jax
import jax.numpy as jnp
from jax import lax
from jax.experimental import pallas as pl
from jax.experimental.pallas import tpu as pltpu

B, H, D, BS = 32, 16, 128, 32
PAGES_LOCAL = 256
SCALE = D ** -0.5
NEG = -1e30


def _partials_body(q_ref, bt_ref, lens_ref, k_hbm, v_hbm,
                   acc_ref, m_ref, l_ref,
                   k_buf, v_buf, sems):
    i = pl.program_id(0)
    my_z = lax.axis_index("z")
    base = my_z * PAGES_LOCAL
    n = lens_ref[i]
    q = q_ref[0, 0]

    def page_step(j, carry):
        m, l, acc = carry
        p_loc = bt_ref[i, j] - base
        owned = jnp.logical_and(p_loc >= 0, p_loc < PAGES_LOCAL)
        pc = jnp.clip(p_loc, 0, PAGES_LOCAL - 1)
        ck = pltpu.make_async_copy(k_hbm.at[pc], k_buf, sems.at[0])
        cv = pltpu.make_async_copy(v_hbm.at[pc], v_buf, sems.at[1])
        ck.start()
        cv.start()
        ck.wait()
        cv.wait()
        k = k_buf[...]
        s = jnp.sum(q[None, :, :] * k, axis=-1) * SCALE
        page_max = jnp.where(owned, jnp.max(s, axis=0, keepdims=True), NEG)
        m_new = jnp.maximum(m, page_max)
        alpha = jnp.exp(m - m_new)
        pexp = jnp.where(owned, jnp.exp(s - m_new), 0.0)
        l_new = alpha * l + jnp.sum(pexp, axis=0, keepdims=True)
        v = v_buf[...]
        pv = jnp.sum(pexp[:, :, None] * v, axis=0)
        acc_new = acc * jnp.reshape(alpha, (H, 1)) + pv
        return m_new, l_new, acc_new

    m0 = jnp.full((1, H), NEG, jnp.float32)
    l0 = jnp.zeros((1, H), jnp.float32)
    a0 = jnp.zeros((H, D), jnp.float32)
    m, l, acc = lax.fori_loop(0, n, page_step, (m0, l0, a0))
    m_ref[pl.ds(i, 1), :] = m
    l_ref[pl.ds(i, 1), :] = l
    acc_ref[0] = acc


def _combine_body(acc_ref, m_ref, l_ref, out_ref,
                  racc_ref, rm_ref, rl_ref, send_sems, recv_sems):
    x = lax.axis_index("x")
    y = lax.axis_index("y")
    z = lax.axis_index("z")
    partner = (x, y, 1 - z)

    bsem = pltpu.get_barrier_semaphore()
    pl.semaphore_signal(bsem, inc=1, device_id=partner,
                        device_id_type=pl.DeviceIdType.MESH)
    pl.semaphore_wait(bsem, 1)

    copies = []
    for idx, (src, dst) in enumerate(
        [(acc_ref, racc_ref), (m_ref, rm_ref), (l_ref, rl_ref)]
    ):
        c = pltpu.make_async_remote_copy(
            src_ref=src, dst_ref=dst,
            send_sem=send_sems.at[idx], recv_sem=recv_sems.at[idx],
            device_id=partner, device_id_type=pl.DeviceIdType.MESH,
        )
        c.start()
        copies.append(c)
    for c in copies:
        c.wait()

    m_a = m_ref[...]
    m_b = rm_ref[...]
    mx = jnp.maximum(m_a, m_b)
    wa = jnp.exp(m_a - mx)
    wb = jnp.exp(m_b - mx)
    denom = l_ref[...] * wa + rl_ref[...] * wb
    num = acc_ref[...] * wa[:, :, None] + racc_ref[...] * wb[:, :, None]
    out_ref[:, 0, :, :] = num / denom[:, :, None]


def kernel(Q, K, V, bt, lens):
    acc, m, l = pl.pallas_call(
        _partials_body,
        grid=(B,),
        in_specs=[
            pl.BlockSpec((1, 1, H, D), lambda i: (i, 0, 0, 0)),
            pl.BlockSpec(memory_space=pltpu.SMEM),
            pl.BlockSpec(memory_space=pltpu.SMEM),
            pl.BlockSpec(memory_space=pl.ANY),
            pl.BlockSpec(memory_space=pl.ANY),
        ],
        out_specs=[
            pl.BlockSpec((1, H, D), lambda i: (i, 0, 0)),
            pl.BlockSpec((B, H), lambda i: (0, 0)),
            pl.BlockSpec((B, H), lambda i: (0, 0)),
        ],
        out_shape=[
            jax.ShapeDtypeStruct((B, H, D), jnp.float32),
            jax.ShapeDtypeStruct((B, H), jnp.float32),
            jax.ShapeDtypeStruct((B, H), jnp.float32),
        ],
        scratch_shapes=[
            pltpu.VMEM((BS, H, D), jnp.float32),
            pltpu.VMEM((BS, H, D), jnp.float32),
            pltpu.SemaphoreType.DMA((2,)),
        ],
    )(Q, bt, lens, K, V)

    return pl.pallas_call(
        _combine_body,
        in_specs=[pl.BlockSpec(memory_space=pltpu.VMEM)] * 3,
        out_specs=pl.BlockSpec(memory_space=pltpu.VMEM),
        out_shape=jax.ShapeDtypeStruct((B, 1, H, D), jnp.float32),
        scratch_shapes=[
            pltpu.VMEM((B, H, D), jnp.float32),
            pltpu.VMEM((B, H), jnp.float32),
            pltpu.VMEM((B, H), jnp.float32),
            pltpu.SemaphoreType.DMA((3,)),
            pltpu.SemaphoreType.DMA((3,)),
        ],
        compiler_params=pltpu.CompilerParams(collective_id=0),
    )(acc, m, l)


# device time: 990517 ns/iter; 5.9186x vs baseline; 5.9186x over previous
import jax
import jax.numpy as jnp
from jax import lax
from jax.experimental import pallas as pl
from jax.experimental.pallas import tpu as pltpu

B, H, D, BS = 32, 16, 128, 32
PAGES_LOCAL = 256
SCALE = D ** -0.5
NEG = -1e30


CH = 8


def _partials_body(q_ref, pages_ref, counts_ref, k_hbm, v_hbm,
                   acc_ref, m_ref, l_ref,
                   k_buf, v_buf, sems):
    i = pl.program_id(0)
    c = counts_ref[i]
    q = q_ref[0, 0]
    n_chunks = (c + CH - 1) // CH

    def chunk_step(t, carry):
        m, l, acc = carry
        c0 = t * CH
        copies = []
        for u in range(CH):
            idx = pages_ref[i, jnp.minimum(c0 + u, 255)]
            ck = pltpu.make_async_copy(
                k_hbm.at[idx], k_buf.at[pl.ds(u * BS, BS)], sems.at[0])
            cv = pltpu.make_async_copy(
                v_hbm.at[idx], v_buf.at[pl.ds(u * BS, BS)], sems.at[1])
            ck.start()
            cv.start()
            copies += [ck, cv]
        for cp in copies:
            cp.wait()
        ks = k_buf[...]
        s = jnp.sum(q[None, :, :] * ks, axis=-1) * SCALE
        rows = lax.broadcasted_iota(jnp.int32, (CH * BS, H), 0)
        valid = (c0 + rows // BS) < c
        s = jnp.where(valid, s, NEG)
        m_new = jnp.maximum(m, jnp.max(s, axis=0, keepdims=True))
        alpha = jnp.exp(m - m_new)
        pexp = jnp.where(valid, jnp.exp(s - m_new), 0.0)
        l_new = alpha * l + jnp.sum(pexp, axis=0, keepdims=True)
        vs = v_buf[...]
        pv = jnp.sum(pexp[:, :, None] * vs, axis=0)
        acc_new = acc * jnp.reshape(alpha, (H, 1)) + pv
        return m_new, l_new, acc_new

    m0 = jnp.full((1, H), NEG, jnp.float32)
    l0 = jnp.zeros((1, H), jnp.float32)
    a0 = jnp.zeros((H, D), jnp.float32)
    m, l, acc = lax.fori_loop(0, n_chunks, chunk_step, (m0, l0, a0))
    m_ref[pl.ds(i, 1), :] = m
    l_ref[pl.ds(i, 1), :] = l
    acc_ref[0] = acc


def _combine_body(acc_ref, m_ref, l_ref, out_ref,
                  racc_ref, rm_ref, rl_ref, send_sems, recv_sems):
    x = lax.axis_index("x")
    y = lax.axis_index("y")
    z = lax.axis_index("z")
    partner = (x, y, 1 - z)

    bsem = pltpu.get_barrier_semaphore()
    pl.semaphore_signal(bsem, inc=1, device_id=partner,
                        device_id_type=pl.DeviceIdType.MESH)
    pl.semaphore_wait(bsem, 1)

    copies = []
    for idx, (src, dst) in enumerate(
        [(acc_ref, racc_ref), (m_ref, rm_ref), (l_ref, rl_ref)]
    ):
        c = pltpu.make_async_remote_copy(
            src_ref=src, dst_ref=dst,
            send_sem=send_sems.at[idx], recv_sem=recv_sems.at[idx],
            device_id=partner, device_id_type=pl.DeviceIdType.MESH,
        )
        c.start()
        copies.append(c)
    for c in copies:
        c.wait()

    m_a = m_ref[...]
    m_b = rm_ref[...]
    mx = jnp.maximum(m_a, m_b)
    wa = jnp.exp(m_a - mx)
    wb = jnp.exp(m_b - mx)
    denom = l_ref[...] * wa + rl_ref[...] * wb
    num = acc_ref[...] * wa[:, :, None] + racc_ref[...] * wb[:, :, None]
    out_ref[:, 0, :, :] = num / denom[:, :, None]


def kernel(Q, K, V, bt, lens):
    base = lax.axis_index("z") * PAGES_LOCAL
    col = jnp.arange(256, dtype=jnp.int32)[None, :]
    loc = bt - base
    owned = (col < lens[:, None]) & (loc >= 0) & (loc < PAGES_LOCAL)
    order = jnp.argsort(jnp.logical_not(owned), axis=1, stable=True)
    pages = jnp.take_along_axis(
        jnp.clip(loc, 0, PAGES_LOCAL - 1), order, axis=1
    ).astype(jnp.int32)
    counts = jnp.sum(owned, axis=1).astype(jnp.int32)

    acc, m, l = pl.pallas_call(
        _partials_body,
        grid=(B,),
        in_specs=[
            pl.BlockSpec((1, 1, H, D), lambda i: (i, 0, 0, 0)),
            pl.BlockSpec(memory_space=pltpu.SMEM),
            pl.BlockSpec(memory_space=pltpu.SMEM),
            pl.BlockSpec(memory_space=pl.ANY),
            pl.BlockSpec(memory_space=pl.ANY),
        ],
        out_specs=[
            pl.BlockSpec((1, H, D), lambda i: (i, 0, 0)),
            pl.BlockSpec((B, H), lambda i: (0, 0)),
            pl.BlockSpec((B, H), lambda i: (0, 0)),
        ],
        out_shape=[
            jax.ShapeDtypeStruct((B, H, D), jnp.float32),
            jax.ShapeDtypeStruct((B, H), jnp.float32),
            jax.ShapeDtypeStruct((B, H), jnp.float32),
        ],
        scratch_shapes=[
            pltpu.VMEM((CH * BS, H, D), jnp.float32),
            pltpu.VMEM((CH * BS, H, D), jnp.float32),
            pltpu.SemaphoreType.DMA((2,)),
        ],
    )(Q, pages, counts, K, V)

    return pl.pallas_call(
        _combine_body,
        in_specs=[pl.BlockSpec(memory_space=pltpu.VMEM)] * 3,
        out_specs=pl.BlockSpec(memory_space=pltpu.VMEM),
        out_shape=jax.ShapeDtypeStruct((B, 1, H, D), jnp.float32),
        scratch_shapes=[
            pltpu.VMEM((B, H, D), jnp.float32),
            pltpu.VMEM((B, H), jnp.float32),
            pltpu.VMEM((B, H), jnp.float32),
            pltpu.SemaphoreType.DMA((3,)),
            pltpu.SemaphoreType.DMA((3,)),
        ],
        compiler_params=pltpu.CompilerParams(collective_id=0),
    )(acc, m, l)


# device time: 568091 ns/iter; 10.3196x vs baseline; 1.7436x over previous
import jax
import jax.numpy as jnp
from jax import lax
from jax.experimental import pallas as pl
from jax.experimental.pallas import tpu as pltpu

B, H, D, BS = 32, 16, 128, 32
PAGES_LOCAL = 256
SCALE = D ** -0.5
NEG = -1e30


CH = 8


def _partials_body(q_ref, pages_ref, counts_ref, k_hbm, v_hbm,
                   acc_ref, m_ref, l_ref,
                   k_buf, v_buf, sems):
    i = pl.program_id(0)
    c = counts_ref[i]
    q = q_ref[0, 0]
    n_chunks = (c + CH - 1) // CH

    def issue_chunk(t, buf):
        c0 = t * CH
        for u in range(CH):
            idx = pages_ref[i, jnp.minimum(c0 + u, 255)]
            pltpu.make_async_copy(
                k_hbm.at[idx], k_buf.at[buf].at[pl.ds(u * BS, BS)],
                sems.at[0, buf]).start()
            pltpu.make_async_copy(
                v_hbm.at[idx], v_buf.at[buf].at[pl.ds(u * BS, BS)],
                sems.at[1, buf]).start()

    def wait_chunk(buf):
        for u in range(CH):
            pltpu.make_async_copy(
                k_hbm.at[0], k_buf.at[buf].at[pl.ds(u * BS, BS)],
                sems.at[0, buf]).wait()
            pltpu.make_async_copy(
                v_hbm.at[0], v_buf.at[buf].at[pl.ds(u * BS, BS)],
                sems.at[1, buf]).wait()

    @pl.when(n_chunks > 0)
    def _():
        issue_chunk(0, 0)

    def chunk_step(t, carry):
        m, l, acc = carry
        c0 = t * CH
        buf = lax.rem(t, 2)

        @pl.when(t + 1 < n_chunks)
        def _():
            issue_chunk(t + 1, 1 - buf)

        wait_chunk(buf)
        ks = k_buf[buf]
        s = jnp.sum(q[None, :, :] * ks, axis=-1) * SCALE
        rows = lax.broadcasted_iota(jnp.int32, (CH * BS, H), 0)
        valid = (c0 + rows // BS) < c
        s = jnp.where(valid, s, NEG)
        m_new = jnp.maximum(m, jnp.max(s, axis=0, keepdims=True))
        alpha = jnp.exp(m - m_new)
        pexp = jnp.where(valid, jnp.exp(s - m_new), 0.0)
        l_new = alpha * l + jnp.sum(pexp, axis=0, keepdims=True)
        vs = v_buf[buf]
        pv = jnp.sum(pexp[:, :, None] * vs, axis=0)
        acc_new = acc * jnp.reshape(alpha, (H, 1)) + pv
        return m_new, l_new, acc_new

    m0 = jnp.full((1, H), NEG, jnp.float32)
    l0 = jnp.zeros((1, H), jnp.float32)
    a0 = jnp.zeros((H, D), jnp.float32)
    m, l, acc = lax.fori_loop(0, n_chunks, chunk_step, (m0, l0, a0))
    m_ref[pl.ds(i, 1), :] = m
    l_ref[pl.ds(i, 1), :] = l
    acc_ref[0] = acc


def _combine_body(acc_ref, m_ref, l_ref, out_ref,
                  racc_ref, rm_ref, rl_ref, send_sems, recv_sems):
    x = lax.axis_index("x")
    y = lax.axis_index("y")
    z = lax.axis_index("z")
    partner = (x, y, 1 - z)

    bsem = pltpu.get_barrier_semaphore()
    pl.semaphore_signal(bsem, inc=1, device_id=partner,
                        device_id_type=pl.DeviceIdType.MESH)
    pl.semaphore_wait(bsem, 1)

    copies = []
    for idx, (src, dst) in enumerate(
        [(acc_ref, racc_ref), (m_ref, rm_ref), (l_ref, rl_ref)]
    ):
        c = pltpu.make_async_remote_copy(
            src_ref=src, dst_ref=dst,
            send_sem=send_sems.at[idx], recv_sem=recv_sems.at[idx],
            device_id=partner, device_id_type=pl.DeviceIdType.MESH,
        )
        c.start()
        copies.append(c)
    for c in copies:
        c.wait()

    m_a = m_ref[...]
    m_b = rm_ref[...]
    mx = jnp.maximum(m_a, m_b)
    wa = jnp.exp(m_a - mx)
    wb = jnp.exp(m_b - mx)
    denom = l_ref[...] * wa + rl_ref[...] * wb
    num = acc_ref[...] * wa[:, :, None] + racc_ref[...] * wb[:, :, None]
    out_ref[:, 0, :, :] = num / denom[:, :, None]


def kernel(Q, K, V, bt, lens):
    base = lax.axis_index("z") * PAGES_LOCAL
    col = jnp.arange(256, dtype=jnp.int32)[None, :]
    loc = bt - base
    owned = (col < lens[:, None]) & (loc >= 0) & (loc < PAGES_LOCAL)
    order = jnp.argsort(jnp.logical_not(owned), axis=1, stable=True)
    pages = jnp.take_along_axis(
        jnp.clip(loc, 0, PAGES_LOCAL - 1), order, axis=1
    ).astype(jnp.int32)
    counts = jnp.sum(owned, axis=1).astype(jnp.int32)

    acc, m, l = pl.pallas_call(
        _partials_body,
        grid=(B,),
        in_specs=[
            pl.BlockSpec((1, 1, H, D), lambda i: (i, 0, 0, 0)),
            pl.BlockSpec(memory_space=pltpu.SMEM),
            pl.BlockSpec(memory_space=pltpu.SMEM),
            pl.BlockSpec(memory_space=pl.ANY),
            pl.BlockSpec(memory_space=pl.ANY),
        ],
        out_specs=[
            pl.BlockSpec((1, H, D), lambda i: (i, 0, 0)),
            pl.BlockSpec((B, H), lambda i: (0, 0)),
            pl.BlockSpec((B, H), lambda i: (0, 0)),
        ],
        out_shape=[
            jax.ShapeDtypeStruct((B, H, D), jnp.float32),
            jax.ShapeDtypeStruct((B, H), jnp.float32),
            jax.ShapeDtypeStruct((B, H), jnp.float32),
        ],
        scratch_shapes=[
            pltpu.VMEM((2, CH * BS, H, D), jnp.float32),
            pltpu.VMEM((2, CH * BS, H, D), jnp.float32),
            pltpu.SemaphoreType.DMA((2, 2)),
        ],
    )(Q, pages, counts, K, V)

    return pl.pallas_call(
        _combine_body,
        in_specs=[pl.BlockSpec(memory_space=pltpu.VMEM)] * 3,
        out_specs=pl.BlockSpec(memory_space=pltpu.VMEM),
        out_shape=jax.ShapeDtypeStruct((B, 1, H, D), jnp.float32),
        scratch_shapes=[
            pltpu.VMEM((B, H, D), jnp.float32),
            pltpu.VMEM((B, H), jnp.float32),
            pltpu.VMEM((B, H), jnp.float32),
            pltpu.SemaphoreType.DMA((3,)),
            pltpu.SemaphoreType.DMA((3,)),
        ],
        compiler_params=pltpu.CompilerParams(collective_id=0),
    )(acc, m, l)


# device time: 201310 ns/iter; 29.1216x vs baseline; 2.8220x over previous
import jax
import jax.numpy as jnp
from jax import lax
from jax.experimental import pallas as pl
from jax.experimental.pallas import tpu as pltpu

B, H, D, BS = 32, 16, 128, 32
PAGES_LOCAL = 256
NB = 256
SCALE = D ** -0.5
NEG = -1e30
CH = 8
BL = B // 4
N_RING = 4


def _partials_body(q_ref, pages_ref, counts_ref, k_hbm, v_hbm,
                   acc_ref, m_ref, l_ref,
                   k_buf, v_buf, sems):
    i = pl.program_id(0)
    c = counts_ref[i]
    q = q_ref[0, 0]
    n_chunks = (c + CH - 1) // CH

    def issue_chunk(t, buf):
        c0 = t * CH
        for u in range(CH):
            idx = pages_ref[i, jnp.minimum(c0 + u, NB - 1)]
            pltpu.make_async_copy(
                k_hbm.at[idx], k_buf.at[buf].at[pl.ds(u * BS, BS)],
                sems.at[0, buf]).start()
            pltpu.make_async_copy(
                v_hbm.at[idx], v_buf.at[buf].at[pl.ds(u * BS, BS)],
                sems.at[1, buf]).start()

    def wait_chunk(buf):
        for u in range(CH):
            pltpu.make_async_copy(
                k_hbm.at[0], k_buf.at[buf].at[pl.ds(u * BS, BS)],
                sems.at[0, buf]).wait()
            pltpu.make_async_copy(
                k_hbm.at[0], v_buf.at[buf].at[pl.ds(u * BS, BS)],
                sems.at[1, buf]).wait()

    @pl.when(n_chunks > 0)
    def _():
        issue_chunk(0, 0)

    def chunk_step(t, carry):
        m, l, acc = carry
        c0 = t * CH
        buf = lax.rem(t, 2)

        @pl.when(t + 1 < n_chunks)
        def _():
            issue_chunk(t + 1, 1 - buf)

        wait_chunk(buf)
        ks = k_buf[buf]
        s = jnp.sum(q[None, :, :] * ks, axis=-1) * SCALE
        rows = lax.broadcasted_iota(jnp.int32, (CH * BS, H), 0)
        valid = (c0 + rows // BS) < c
        s = jnp.where(valid, s, NEG)
        m_new = jnp.maximum(m, jnp.max(s, axis=0, keepdims=True))
        alpha = jnp.exp(m - m_new)
        pexp = jnp.where(valid, jnp.exp(s - m_new), 0.0)
        l_new = alpha * l + jnp.sum(pexp, axis=0, keepdims=True)
        vs = v_buf[buf]
        pv = jnp.sum(pexp[:, :, None] * vs, axis=0)
        acc_new = acc * jnp.reshape(alpha, (H, 1)) + pv
        return m_new, l_new, acc_new

    m0 = jnp.full((1, H), NEG, jnp.float32)
    l0 = jnp.zeros((1, H), jnp.float32)
    a0 = jnp.zeros((H, D), jnp.float32)
    m, l, acc = lax.fori_loop(0, n_chunks, chunk_step, (m0, l0, a0))
    m_ref[pl.ds(i, 1), :] = m
    l_ref[pl.ds(i, 1), :] = l
    acc_ref[0] = acc


def _combine_body(acc_ref, m_ref, l_ref, out_ref,
                  racc_ref, rm_ref, rl_ref,
                  zsend_sems, zrecv_sems, gsend_sems, grecv_sems):
    x = lax.axis_index("x")
    y = lax.axis_index("y")
    z = lax.axis_index("z")
    partner = (x, y, 1 - z)
    p1 = (x, 1 - y, z)
    p2 = (1 - x, y, z)
    blk = 2 * x + y

    bsem = pltpu.get_barrier_semaphore()
    for nbr in (partner, p1, p2):
        pl.semaphore_signal(bsem, inc=1, device_id=nbr,
                            device_id_type=pl.DeviceIdType.MESH)
    pl.semaphore_wait(bsem, 3)

    copies = []
    for idx, (src, dst) in enumerate(
        [(acc_ref, racc_ref), (m_ref, rm_ref), (l_ref, rl_ref)]
    ):
        cp = pltpu.make_async_remote_copy(
            src_ref=src, dst_ref=dst,
            send_sem=zsend_sems.at[idx], recv_sem=zrecv_sems.at[idx],
            device_id=partner, device_id_type=pl.DeviceIdType.MESH,
        )
        cp.start()
        copies.append(cp)
    for cp in copies:
        cp.wait()

    m_a = m_ref[...]
    m_b = rm_ref[...]
    mx = jnp.maximum(m_a, m_b)
    wa = jnp.exp(m_a - mx)
    wb = jnp.exp(m_b - mx)
    denom = l_ref[...] * wa + rl_ref[...] * wb
    num = acc_ref[...] * wa[:, :, None] + racc_ref[...] * wb[:, :, None]
    comb = num / denom[:, :, None]

    out_ref[pl.ds(blk * BL, BL), 0] = comb

    ex1 = pltpu.make_async_remote_copy(
        src_ref=out_ref.at[pl.ds(blk * BL, BL)],
        dst_ref=out_ref.at[pl.ds(blk * BL, BL)],
        send_sem=gsend_sems.at[0], recv_sem=grecv_sems.at[0],
        device_id=p1, device_id_type=pl.DeviceIdType.MESH,
    )
    ex1.start()
    ex1.wait()
    half = x * (2 * BL)
    ex2 = pltpu.make_async_remote_copy(
        src_ref=out_ref.at[pl.ds(half, 2 * BL)],
        dst_ref=out_ref.at[pl.ds(half, 2 * BL)],
        send_sem=gsend_sems.at[1], recv_sem=grecv_sems.at[1],
        device_id=p2, device_id_type=pl.DeviceIdType.MESH,
    )
    ex2.start()
    ex2.wait()


def kernel(Q, K, V, bt, lens):
    x = lax.axis_index("x")
    y = lax.axis_index("y")
    i0 = (2 * x + y) * BL
    Qs = lax.dynamic_slice(Q, (i0, 0, 0, 0), (BL, 1, H, D))
    bts = lax.dynamic_slice(bt, (i0, 0), (BL, NB))
    lens_s = lax.dynamic_slice(lens, (i0,), (BL,))

    base = lax.axis_index("z") * PAGES_LOCAL
    col = jnp.arange(NB, dtype=jnp.int32)[None, :]
    loc = bts - base
    owned = (col < lens_s[:, None]) & (loc >= 0) & (loc < PAGES_LOCAL)
    order = jnp.argsort(jnp.logical_not(owned), axis=1, stable=True)
    pages = jnp.take_along_axis(
        jnp.clip(loc, 0, PAGES_LOCAL - 1), order, axis=1
    ).astype(jnp.int32)
    counts = jnp.sum(owned, axis=1).astype(jnp.int32)

    acc, m, l = pl.pallas_call(
        _partials_body,
        grid=(BL,),
        in_specs=[
            pl.BlockSpec((1, 1, H, D), lambda i: (i, 0, 0, 0)),
            pl.BlockSpec(memory_space=pltpu.SMEM),
            pl.BlockSpec(memory_space=pltpu.SMEM),
            pl.BlockSpec(memory_space=pl.ANY),
            pl.BlockSpec(memory_space=pl.ANY),
        ],
        out_specs=[
            pl.BlockSpec((1, H, D), lambda i: (i, 0, 0)),
            pl.BlockSpec((BL, H), lambda i: (0, 0)),
            pl.BlockSpec((BL, H), lambda i: (0, 0)),
        ],
        out_shape=[
            jax.ShapeDtypeStruct((BL, H, D), jnp.float32),
            jax.ShapeDtypeStruct((BL, H), jnp.float32),
            jax.ShapeDtypeStruct((BL, H), jnp.float32),
        ],
        scratch_shapes=[
            pltpu.VMEM((2, CH * BS, H, D), jnp.float32),
            pltpu.VMEM((2, CH * BS, H, D), jnp.float32),
            pltpu.SemaphoreType.DMA((2, 2)),
        ],
    )(Qs, pages, counts, K, V)

    return pl.pallas_call(
        _combine_body,
        in_specs=[pl.BlockSpec(memory_space=pltpu.VMEM)] * 3,
        out_specs=pl.BlockSpec(memory_space=pltpu.VMEM),
        out_shape=jax.ShapeDtypeStruct((B, 1, H, D), jnp.float32),
        scratch_shapes=[
            pltpu.VMEM((BL, H, D), jnp.float32),
            pltpu.VMEM((BL, H), jnp.float32),
            pltpu.VMEM((BL, H), jnp.float32),
            pltpu.SemaphoreType.DMA((3,)),
            pltpu.SemaphoreType.DMA((3,)),
            pltpu.SemaphoreType.DMA((2,)),
            pltpu.SemaphoreType.DMA((2,)),
        ],
        compiler_params=pltpu.CompilerParams(collective_id=0),
    )(acc, m, l)
